# baseline (speedup 1.0000x reference)
import jax
import jax.numpy as jnp
from jax import lax
from jax.experimental import pallas as pl
from jax.experimental.pallas import tpu as pltpu

_BLOCK_M = 1024


def _router_block(x_ref, wb_ref, o_ref):
    e = o_ref.shape[1]
    w = wb_ref[:e, :]
    b = wb_ref[e:e + 1, :e]
    o_ref[...] = (
        lax.dot_general(
            x_ref[...],
            w,
            (((1,), (1,)), ((), ())),
            preferred_element_type=jnp.float32,
        )
        + b
    )


def kernel(hidden_states, weight, bias):
    b, s, h = hidden_states.shape
    e = weight.shape[0]
    m = b * s
    x = hidden_states.reshape(m, h)
    wb = jnp.zeros((e + 8, h), jnp.float32)
    wb = wb.at[:e, :].set(weight)
    wb = wb.at[e, :e].set(bias)

    block_m = min(_BLOCK_M, m)
    grid = (m // block_m,)
    out = pl.pallas_call(
        _router_block,
        grid=grid,
        in_specs=[
            pl.BlockSpec((block_m, h), lambda i: (i, 0)),
            pl.BlockSpec((e + 8, h), lambda i: (0, 0)),
        ],
        out_specs=pl.BlockSpec((block_m, e), lambda i: (i, 0)),
        out_shape=jax.ShapeDtypeStruct((m, e), jnp.float32),
        compiler_params=pltpu.CompilerParams(
            dimension_semantics=("arbitrary",),
        ),
    )(x, wb)
    return out


# speedup vs baseline: 1.0194x; 1.0194x over previous
import jax
import jax.numpy as jnp
from jax import lax
from jax.experimental import pallas as pl
from jax.experimental.pallas import tpu as pltpu

_BLOCK_M = 1024


def _router_block(x_ref, w_ref, b_ref, o_ref):
    o_ref[...] = (
        lax.dot_general(
            x_ref[...],
            w_ref[...],
            (((1,), (1,)), ((), ())),
            preferred_element_type=jnp.float32,
        )
        + b_ref[...]
    )


def kernel(hidden_states, weight, bias):
    b, s, h = hidden_states.shape
    e = weight.shape[0]
    m = b * s
    x = hidden_states.reshape(m, h)
    bias2 = bias.reshape(1, e)

    block_m = min(_BLOCK_M, m)
    grid = (m // block_m,)
    out = pl.pallas_call(
        _router_block,
        grid=grid,
        in_specs=[
            pl.BlockSpec((block_m, h), lambda i: (i, 0)),
            pl.BlockSpec((e, h), lambda i: (0, 0)),
            pl.BlockSpec((1, e), lambda i: (0, 0)),
        ],
        out_specs=pl.BlockSpec((block_m, e), lambda i: (i, 0)),
        out_shape=jax.ShapeDtypeStruct((m, e), jnp.float32),
        compiler_params=pltpu.CompilerParams(
            dimension_semantics=("arbitrary",),
        ),
    )(x, weight, bias2)
    return out
